# Initial kernel scaffold; baseline (speedup 1.0000x reference)
import jax
import jax.numpy as jnp
from jax.experimental import pallas as pl


def kernel(x):
    raise NotImplementedError("write your pallas kernel here")


# speedup vs baseline: 39.7133x; 39.7133x over previous
import functools

import jax
import jax.numpy as jnp
from jax.experimental import pallas as pl
from jax.experimental.pallas import tpu as pltpu

_POS_INF_KEY = 0x7F800000
_N_PASSES = 18
_N_CHUNKS = 16


def _select_body(k, x_ref, t_ref, st_ref, cnt_ref):
    p = pl.program_id(0)
    c = pl.program_id(1)
    n_chunks = pl.num_programs(1)
    n_passes = pl.num_programs(0)

    @pl.when(jnp.logical_and(p == 0, c == 0))
    def _init():
        st_ref[0] = jnp.int32(0)
        st_ref[1] = jnp.int32(_POS_INF_KEY)

    lo = st_ref[0]
    hi = st_ref[1]
    mid2 = lo + (hi - lo) // 2
    mid1 = lo + (mid2 - lo) // 2
    mid3 = mid2 + (hi - mid2) // 2

    xi = jax.lax.bitcast_convert_type(x_ref[...], jnp.int32)
    c1 = jnp.sum((xi >= mid1).astype(jnp.int32))
    c2 = jnp.sum((xi >= mid2).astype(jnp.int32))
    c3 = jnp.sum((xi >= mid3).astype(jnp.int32))

    @pl.when(c == 0)
    def _reset():
        cnt_ref[0] = c1
        cnt_ref[1] = c2
        cnt_ref[2] = c3

    @pl.when(c != 0)
    def _acc():
        cnt_ref[0] += c1
        cnt_ref[1] += c2
        cnt_ref[2] += c3

    @pl.when(c == n_chunks - 1)
    def _update():
        t1 = cnt_ref[0] < k
        t2 = cnt_ref[1] < k
        t3 = cnt_ref[2] < k
        new_lo = jnp.where(t1, lo, jnp.where(t2, mid1, jnp.where(t3, mid2, mid3)))
        new_hi = jnp.where(t1, mid1, jnp.where(t2, mid2, jnp.where(t3, mid3, hi)))
        st_ref[0] = new_lo
        st_ref[1] = new_hi

        @pl.when(p == n_passes - 1)
        def _emit():
            t_ref[0] = new_lo


def _mask_body(x_ref, t_ref, o_ref):
    t = t_ref[0]
    xs = x_ref[...]
    keys = jax.lax.bitcast_convert_type(xs, jnp.int32)
    o_ref[...] = jnp.where(keys >= t, xs, 0.0)


def _build_calls(b, d, interpret=False):
    k = min(32 * b, b * d)
    n_chunks = min(_N_CHUNKS, b)
    rows = b // n_chunks
    select = pl.pallas_call(
        functools.partial(_select_body, k),
        grid=(_N_PASSES, n_chunks),
        in_specs=[pl.BlockSpec((rows, d), lambda p, c: (c, 0))],
        out_specs=pl.BlockSpec(memory_space=pltpu.SMEM),
        out_shape=jax.ShapeDtypeStruct((1,), jnp.int32),
        scratch_shapes=[pltpu.SMEM((2,), jnp.int32), pltpu.SMEM((3,), jnp.int32)],
        interpret=interpret,
    )
    mask = pl.pallas_call(
        _mask_body,
        grid=(n_chunks,),
        in_specs=[
            pl.BlockSpec((rows, d), lambda c: (c, 0)),
            pl.BlockSpec(memory_space=pltpu.SMEM),
        ],
        out_specs=pl.BlockSpec((rows, d), lambda c: (c, 0)),
        out_shape=jax.ShapeDtypeStruct((b, d), jnp.float32),
        interpret=interpret,
    )
    return select, mask


def kernel(x):
    b, d = x.shape
    select, mask = _build_calls(b, d)
    t = select(x)
    return mask(x, t)
